# baseline (speedup 1.0000x reference)
import jax
import jax.numpy as jnp
from jax import lax
from jax.experimental import pallas as pl
from jax.experimental.pallas import tpu as pltpu
from jax.experimental.pallas import tpu_sc as plsc

N = 10000
NI = 320000
D = 128
DH = D // 2
G = 64
CW = 16

NC = 2
NS = 16
CHUNK = 128
N_CHUNKS = 160
PER_TILE = CHUNK * N_CHUNKS
NI_PAD = PER_TILE * NS
NP = 10240
PAD_ROW = N
ROWS_PER_TILE = NP // NS

_MESH = plsc.VectorSubcoreMesh(core_axis_name="c", subcore_axis_name="s")

_f32 = jnp.float32


def _layer_body(tabL, tabR, srcA, dstA, srcB, dstB, zeros_nd, zeros_cw,
                out, cnt_out,
                sv0, sv1, sv2, sv3, dv0, dv1, dv2, dv3, rv0, rv1,
                cnt_v, ones_v,
                he_sh, out_sh, cd_sh,
                si0, si1, si2, si3, sg0, sg1, ss0, ss1):
    src_v = (sv0, sv1, sv2, sv3)
    dst_v = (dv0, dv1, dv2, dv3)
    rows_v = (rv0, rv1)
    sem_i = (si0, si1, si2, si3)
    sem_g = (sg0, sg1)
    sem_s = (ss0, ss1)

    cid = lax.axis_index("c")
    sid = lax.axis_index("s")

    @pl.when(sid == 0)
    def _():
        pltpu.sync_copy(zeros_nd, he_sh)
        pltpu.sync_copy(zeros_nd, out_sh)
        pltpu.sync_copy(zeros_cw, cd_sh)

    for r in range(CHUNK):
        ones_v[r, :] = jnp.ones((CW,), _f32)

    plsc.subcore_barrier()

    def run_pipeline(src, dst, acc_sh, gather_issue, gather_wait):
        def issue_idx(j, c):
            base = sid * PER_TILE + c * CHUNK
            pltpu.async_copy(src.at[pl.ds(base, CHUNK)], src_v[j], sem_i[j])
            pltpu.async_copy(dst.at[pl.ds(base, CHUNK)], dst_v[j], sem_i[j])

        def wait_idx(j):
            pltpu.make_async_copy(
                src.at[pl.ds(0, CHUNK)], src_v[j], sem_i[j]).wait()
            pltpu.make_async_copy(
                dst.at[pl.ds(0, CHUNK)], dst_v[j], sem_i[j]).wait()

        def issue_scatter(j, b):
            pltpu.async_copy(rows_v[b], acc_sh.at[dst_v[j]], sem_s[b],
                             add=True)
            pltpu.async_copy(ones_v, cd_sh.at[dst_v[j]], sem_s[b], add=True)

        def wait_scatter(j, b):
            pltpu.make_async_copy(
                rows_v[b], acc_sh.at[dst_v[j]], sem_s[b]).wait()
            pltpu.make_async_copy(
                ones_v, cd_sh.at[dst_v[j]], sem_s[b]).wait()

        issue_idx(0, 0)
        issue_idx(1, 1)

        def body(s, carry):
            for j in range(4):
                c = 4 * s + j
                b = j % 2
                wait_idx(j)

                @pl.when(c >= 2)
                def _(j=j, b=b):
                    wait_scatter((j + 2) % 4, b)

                @pl.when(c + 2 < N_CHUNKS)
                def _(j=j, c=c):
                    issue_idx((j + 2) % 4, c + 2)

                gather_issue(j, b)

                @pl.when(c >= 1)
                def _(j=j, b=b):
                    gather_wait((j + 3) % 4, 1 - b)
                    issue_scatter((j + 3) % 4, 1 - b)

            return carry

        lax.fori_loop(0, N_CHUNKS // 4, body, 0)

        j_last = (N_CHUNKS - 1) % 4
        b_last = (N_CHUNKS - 1) % 2
        gather_wait(j_last, b_last)
        issue_scatter(j_last, b_last)
        wait_scatter((N_CHUNKS - 2) % 4, (N_CHUNKS - 2) % 2)
        wait_scatter(j_last, b_last)

    def gatherA_issue(j, b):
        @pl.when(cid == 0)
        def _():
            pltpu.async_copy(tabL.at[src_v[j]], rows_v[b], sem_g[b])

        @pl.when(cid == 1)
        def _():
            pltpu.async_copy(tabR.at[src_v[j]], rows_v[b], sem_g[b])

    def gatherA_wait(j, b):
        pltpu.make_async_copy(tabL.at[src_v[j]], rows_v[b], sem_g[b]).wait()

    run_pipeline(srcA, dstA, he_sh, gatherA_issue, gatherA_wait)
    plsc.subcore_barrier()

    r0 = sid * ROWS_PER_TILE

    def scale_chunk(k, carry):
        pltpu.sync_copy(he_sh.at[pl.ds(r0 + k * CHUNK, CHUNK)], rows_v[0])
        pltpu.sync_copy(cd_sh.at[pl.ds(r0 + k * CHUNK, CHUNK)], cnt_v)

        def row(r, c2):
            cnt = cnt_v[r, :]
            inv = jnp.where(cnt > 0.0, 1.0 / cnt, 0.0)
            for c in range(DH // 16):
                rows_v[0][r, pl.ds(c * 16, 16)] = (
                    rows_v[0][r, pl.ds(c * 16, 16)] * inv)
            return c2

        lax.fori_loop(0, CHUNK, row, 0)
        pltpu.sync_copy(rows_v[0], he_sh.at[pl.ds(r0 + k * CHUNK, CHUNK)])
        return carry

    lax.fori_loop(0, ROWS_PER_TILE // CHUNK, scale_chunk, 0)
    plsc.subcore_barrier()

    @pl.when(sid == 0)
    def _():
        pltpu.sync_copy(zeros_cw, cd_sh)

    plsc.subcore_barrier()

    def gatherB_issue(j, b):
        pltpu.async_copy(he_sh.at[src_v[j]], rows_v[b], sem_g[b])

    def gatherB_wait(j, b):
        pltpu.make_async_copy(he_sh.at[src_v[j]], rows_v[b], sem_g[b]).wait()

    run_pipeline(srcB, dstB, out_sh, gatherB_issue, gatherB_wait)
    plsc.subcore_barrier()

    def wb(k, carry):
        pltpu.sync_copy(out_sh.at[pl.ds(r0 + k * CHUNK, CHUNK)], rows_v[0])
        pltpu.sync_copy(rows_v[0],
                        out.at[pl.ds(cid * NP + r0 + k * CHUNK, CHUNK)])
        pltpu.sync_copy(cd_sh.at[pl.ds(r0 + k * CHUNK, CHUNK)], cnt_v)
        pltpu.sync_copy(cnt_v,
                        cnt_out.at[pl.ds(cid * NP + r0 + k * CHUNK, CHUNK)])
        return carry

    lax.fori_loop(0, ROWS_PER_TILE // CHUNK, wb, 0)


_layer = pl.kernel(
    _layer_body,
    out_type=(
        jax.ShapeDtypeStruct((NC * NP, DH), _f32),
        jax.ShapeDtypeStruct((NC * NP, CW), _f32),
    ),
    mesh=_MESH,
    scratch_types=(
        [pltpu.VMEM((CHUNK,), jnp.int32)] * 8
        + [pltpu.VMEM((CHUNK, DH), _f32)] * 2
        + [pltpu.VMEM((CHUNK, CW), _f32)] * 2
        + [pltpu.VMEM_SHARED((NP, DH), _f32)] * 2
        + [pltpu.VMEM_SHARED((NP, CW), _f32)]
        + [pltpu.SemaphoreType.DMA] * 8
    ),
    compiler_params=pltpu.CompilerParams(use_tc_tiling_on_sc=False),
)


_RB = 1000
_NB = N // _RB


def _tc_matmul(x, W):
    def body(x_ref, w_ref, l_ref, r_ref):
        res = jnp.dot(x_ref[...], w_ref[...], preferred_element_type=_f32)
        l_ref[...] = res[:, :DH]
        r_ref[...] = res[:, DH:]

    return pl.pallas_call(
        body,
        grid=(_NB,),
        in_specs=[pl.BlockSpec((_RB, D), lambda i: (i, 0)),
                  pl.BlockSpec((D, D), lambda i: (0, 0))],
        out_specs=[pl.BlockSpec((_RB, DH), lambda i: (i, 0)),
                   pl.BlockSpec((_RB, DH), lambda i: (i, 0))],
        out_shape=[jax.ShapeDtypeStruct((N, DH), _f32),
                   jax.ShapeDtypeStruct((N, DH), _f32)],
    )(x, W)


def _tc_combine(partials, cnts, bias, relu=True):
    p3 = partials.reshape(NC, NP, DH)
    c3 = cnts.reshape(NC, NP, CW)

    def body(p_ref, c_ref, b_ref, o_ref):
        s = jnp.concatenate([p_ref[0], p_ref[1]], axis=1)
        cnt = c_ref[0, :, 0:1]
        inv = jnp.where(cnt > 0.0, 1.0 / cnt, 0.0)
        r = s * inv + b_ref[...]
        if relu:
            r = jnp.maximum(r, 0.0)
        o_ref[...] = r

    return pl.pallas_call(
        body,
        grid=(_NB,),
        in_specs=[pl.BlockSpec((NC, _RB, DH), lambda i: (0, i, 0)),
                  pl.BlockSpec((NC, _RB, CW), lambda i: (0, i, 0)),
                  pl.BlockSpec((1, D), lambda i: (0, 0))],
        out_specs=pl.BlockSpec((_RB, D), lambda i: (i, 0)),
        out_shape=jax.ShapeDtypeStruct((N, D), _f32),
    )(p3, c3, bias.reshape(1, D))


def _tc_pool(h, batch3d):
    def body(h_ref, b_ref, o_ref, sums, cnts):
        i = pl.program_id(0)

        @pl.when(i == 0)
        def _():
            sums[...] = jnp.zeros_like(sums)
            cnts[...] = jnp.zeros_like(cnts)

        b = b_ref[0, 0, :]
        mask = (b[:, None] == lax.broadcasted_iota(jnp.int32, (_RB, G), 1)
                ).astype(_f32)
        sums[...] += lax.dot_general(mask, h_ref[...],
                                     (((0,), (0,)), ((), ())),
                                     preferred_element_type=_f32)
        cnts[...] += jnp.broadcast_to(jnp.sum(mask, axis=0)[:, None], (G, D))

        @pl.when(i == _NB - 1)
        def _():
            o_ref[...] = sums[...] / jnp.maximum(cnts[...], 1.0)

    return pl.pallas_call(
        body,
        grid=(_NB,),
        in_specs=[pl.BlockSpec((_RB, D), lambda i: (i, 0)),
                  pl.BlockSpec((1, 1, _RB), lambda i: (i, 0, 0))],
        out_specs=pl.BlockSpec((G, D), lambda i: (0, 0)),
        out_shape=jax.ShapeDtypeStruct((G, D), _f32),
        scratch_shapes=[pltpu.VMEM((G, D), _f32), pltpu.VMEM((G, D), _f32)],
    )(h, batch3d)


def kernel(x, hyperedge_index, batch, W1, b1, W2, b2):
    node_idx = hyperedge_index[0].astype(jnp.int32)
    edge_idx = hyperedge_index[1].astype(jnp.int32)
    batch3d = batch.astype(jnp.int32).reshape(_NB, 1, _RB)

    n_pad = NI_PAD - NI
    src_pad = jnp.arange(n_pad, dtype=jnp.int32) % N
    dst_pad = PAD_ROW + (jnp.arange(n_pad, dtype=jnp.int32) % (NP - N))
    node_src = jnp.concatenate([node_idx, src_pad])
    node_dst = jnp.concatenate([node_idx, dst_pad])
    edge_src = jnp.concatenate([edge_idx, src_pad])
    edge_dst = jnp.concatenate([edge_idx, dst_pad])

    zeros_nd = jnp.zeros((NP, DH), _f32)
    zeros_cw = jnp.zeros((NP, CW), _f32)

    xwL, xwR = _tc_matmul(x, W1)
    outP, cntD = _layer(xwL, xwR, node_src, edge_dst, edge_src, node_dst,
                        zeros_nd, zeros_cw)
    h = _tc_combine(outP, cntD, b1)

    xwL, xwR = _tc_matmul(h, W2)
    outP, cntD = _layer(xwL, xwR, node_src, edge_dst, edge_src, node_dst,
                        zeros_nd, zeros_cw)
    h = _tc_combine(outP, cntD, b2)

    return _tc_pool(h, batch3d)


# speedup vs baseline: 1.0288x; 1.0288x over previous
import jax
import jax.numpy as jnp
from jax import lax
from jax.experimental import pallas as pl
from jax.experimental.pallas import tpu as pltpu
from jax.experimental.pallas import tpu_sc as plsc

N = 10000
NI = 320000
D = 128
DH = D // 2
G = 64
CW = 16

NC = 2
NS = 16
CHUNK = 128
N_CHUNKS = 160
PER_TILE = CHUNK * N_CHUNKS
NI_PAD = PER_TILE * NS
NP = 10240
PAD_ROW = N
ROWS_PER_TILE = NP // NS

_MESH = plsc.VectorSubcoreMesh(core_axis_name="c", subcore_axis_name="s")

_f32 = jnp.float32


def _layer_body(tabL, tabR, srcA, dstA, srcB, dstB, zeros_nd, zeros_cw,
                out, cnt_out, *rest):
    src_v = rest[0:8]
    dst_v = rest[8:16]
    rows_v = rest[16:20]
    cnt_v, ones_v = rest[20:22]
    he_sh, out_sh, cd_sh = rest[22:25]
    sem_i = rest[25:33]
    sem_g = rest[33:37]
    sem_s = rest[37:41]

    cid = lax.axis_index("c")
    sid = lax.axis_index("s")

    @pl.when(sid == 0)
    def _():
        pltpu.sync_copy(zeros_nd, he_sh)
        pltpu.sync_copy(zeros_nd, out_sh)
        pltpu.sync_copy(zeros_cw, cd_sh)

    for r in range(CHUNK):
        ones_v[r, :] = jnp.ones((CW,), _f32)

    plsc.subcore_barrier()

    def run_pipeline(src, dst, acc_sh, gather_issue, gather_wait):
        def issue_idx(j, c):
            base = sid * PER_TILE + c * CHUNK
            pltpu.async_copy(src.at[pl.ds(base, CHUNK)], src_v[j], sem_i[j])
            pltpu.async_copy(dst.at[pl.ds(base, CHUNK)], dst_v[j], sem_i[j])

        def wait_idx(j):
            pltpu.make_async_copy(
                src.at[pl.ds(0, CHUNK)], src_v[j], sem_i[j]).wait()
            pltpu.make_async_copy(
                dst.at[pl.ds(0, CHUNK)], dst_v[j], sem_i[j]).wait()

        def issue_scatter(j, b):
            pltpu.async_copy(rows_v[b], acc_sh.at[dst_v[j]], sem_s[b],
                             add=True)
            pltpu.async_copy(ones_v, cd_sh.at[dst_v[j]], sem_s[b], add=True)

        def wait_scatter(j, b):
            pltpu.make_async_copy(
                rows_v[b], acc_sh.at[dst_v[j]], sem_s[b]).wait()
            pltpu.make_async_copy(
                ones_v, cd_sh.at[dst_v[j]], sem_s[b]).wait()

        for c in range(4):
            issue_idx(c, c)

        def body(s, carry):
            for j in range(8):
                c = 8 * s + j
                b = j % 4
                wait_idx(j)

                @pl.when(c >= 4)
                def _(j=j, b=b):
                    wait_scatter((j + 4) % 8, b)

                @pl.when(c + 4 < N_CHUNKS)
                def _(j=j, c=c):
                    issue_idx((j + 4) % 8, c + 4)

                gather_issue(j, b)

                @pl.when(c >= 1)
                def _(j=j, b=b):
                    gather_wait((j + 7) % 8, (b + 3) % 4)
                    issue_scatter((j + 7) % 8, (b + 3) % 4)

            return carry

        lax.fori_loop(0, N_CHUNKS // 8, body, 0)

        j_last = (N_CHUNKS - 1) % 8
        b_last = (N_CHUNKS - 1) % 4
        gather_wait(j_last, b_last)
        issue_scatter(j_last, b_last)
        for c in range(N_CHUNKS - 4, N_CHUNKS):
            wait_scatter(c % 8, c % 4)

    def gatherA_issue(j, b):
        @pl.when(cid == 0)
        def _():
            pltpu.async_copy(tabL.at[src_v[j]], rows_v[b], sem_g[b])

        @pl.when(cid == 1)
        def _():
            pltpu.async_copy(tabR.at[src_v[j]], rows_v[b], sem_g[b])

    def gatherA_wait(j, b):
        pltpu.make_async_copy(tabL.at[src_v[j]], rows_v[b], sem_g[b]).wait()

    run_pipeline(srcA, dstA, he_sh, gatherA_issue, gatherA_wait)
    plsc.subcore_barrier()

    r0 = sid * ROWS_PER_TILE

    def scale_chunk(k, carry):
        pltpu.sync_copy(he_sh.at[pl.ds(r0 + k * CHUNK, CHUNK)], rows_v[0])
        pltpu.sync_copy(cd_sh.at[pl.ds(r0 + k * CHUNK, CHUNK)], cnt_v)

        def row(r, c2):
            cnt = cnt_v[r, :]
            inv = jnp.where(cnt > 0.0, 1.0 / cnt, 0.0)
            for c in range(DH // 16):
                rows_v[0][r, pl.ds(c * 16, 16)] = (
                    rows_v[0][r, pl.ds(c * 16, 16)] * inv)
            return c2

        lax.fori_loop(0, CHUNK, row, 0)
        pltpu.sync_copy(rows_v[0], he_sh.at[pl.ds(r0 + k * CHUNK, CHUNK)])
        return carry

    lax.fori_loop(0, ROWS_PER_TILE // CHUNK, scale_chunk, 0)
    plsc.subcore_barrier()

    @pl.when(sid == 0)
    def _():
        pltpu.sync_copy(zeros_cw, cd_sh)

    plsc.subcore_barrier()

    def gatherB_issue(j, b):
        pltpu.async_copy(he_sh.at[src_v[j]], rows_v[b], sem_g[b])

    def gatherB_wait(j, b):
        pltpu.make_async_copy(he_sh.at[src_v[j]], rows_v[b], sem_g[b]).wait()

    run_pipeline(srcB, dstB, out_sh, gatherB_issue, gatherB_wait)
    plsc.subcore_barrier()

    def wb(k, carry):
        pltpu.sync_copy(out_sh.at[pl.ds(r0 + k * CHUNK, CHUNK)], rows_v[0])
        pltpu.sync_copy(rows_v[0],
                        out.at[pl.ds(cid * NP + r0 + k * CHUNK, CHUNK)])
        pltpu.sync_copy(cd_sh.at[pl.ds(r0 + k * CHUNK, CHUNK)], cnt_v)
        pltpu.sync_copy(cnt_v,
                        cnt_out.at[pl.ds(cid * NP + r0 + k * CHUNK, CHUNK)])
        return carry

    lax.fori_loop(0, ROWS_PER_TILE // CHUNK, wb, 0)


_layer = pl.kernel(
    _layer_body,
    out_type=(
        jax.ShapeDtypeStruct((NC * NP, DH), _f32),
        jax.ShapeDtypeStruct((NC * NP, CW), _f32),
    ),
    mesh=_MESH,
    scratch_types=(
        [pltpu.VMEM((CHUNK,), jnp.int32)] * 16
        + [pltpu.VMEM((CHUNK, DH), _f32)] * 4
        + [pltpu.VMEM((CHUNK, CW), _f32)] * 2
        + [pltpu.VMEM_SHARED((NP, DH), _f32)] * 2
        + [pltpu.VMEM_SHARED((NP, CW), _f32)]
        + [pltpu.SemaphoreType.DMA] * 16
    ),
    compiler_params=pltpu.CompilerParams(use_tc_tiling_on_sc=False),
)


_RB = 1000
_NB = N // _RB


def _tc_matmul(x, W):
    def body(x_ref, w_ref, l_ref, r_ref):
        res = jnp.dot(x_ref[...], w_ref[...], preferred_element_type=_f32)
        l_ref[...] = res[:, :DH]
        r_ref[...] = res[:, DH:]

    return pl.pallas_call(
        body,
        grid=(_NB,),
        in_specs=[pl.BlockSpec((_RB, D), lambda i: (i, 0)),
                  pl.BlockSpec((D, D), lambda i: (0, 0))],
        out_specs=[pl.BlockSpec((_RB, DH), lambda i: (i, 0)),
                   pl.BlockSpec((_RB, DH), lambda i: (i, 0))],
        out_shape=[jax.ShapeDtypeStruct((N, DH), _f32),
                   jax.ShapeDtypeStruct((N, DH), _f32)],
    )(x, W)


def _tc_combine(partials, cnts, bias, relu=True):
    p3 = partials.reshape(NC, NP, DH)
    c3 = cnts.reshape(NC, NP, CW)

    def body(p_ref, c_ref, b_ref, o_ref):
        s = jnp.concatenate([p_ref[0], p_ref[1]], axis=1)
        cnt = c_ref[0, :, 0:1]
        inv = jnp.where(cnt > 0.0, 1.0 / cnt, 0.0)
        r = s * inv + b_ref[...]
        if relu:
            r = jnp.maximum(r, 0.0)
        o_ref[...] = r

    return pl.pallas_call(
        body,
        grid=(_NB,),
        in_specs=[pl.BlockSpec((NC, _RB, DH), lambda i: (0, i, 0)),
                  pl.BlockSpec((NC, _RB, CW), lambda i: (0, i, 0)),
                  pl.BlockSpec((1, D), lambda i: (0, 0))],
        out_specs=pl.BlockSpec((_RB, D), lambda i: (i, 0)),
        out_shape=jax.ShapeDtypeStruct((N, D), _f32),
    )(p3, c3, bias.reshape(1, D))


def _tc_pool(h, batch3d):
    def body(h_ref, b_ref, o_ref, sums, cnts):
        i = pl.program_id(0)

        @pl.when(i == 0)
        def _():
            sums[...] = jnp.zeros_like(sums)
            cnts[...] = jnp.zeros_like(cnts)

        b = b_ref[0, 0, :]
        mask = (b[:, None] == lax.broadcasted_iota(jnp.int32, (_RB, G), 1)
                ).astype(_f32)
        sums[...] += lax.dot_general(mask, h_ref[...],
                                     (((0,), (0,)), ((), ())),
                                     preferred_element_type=_f32)
        cnts[...] += jnp.broadcast_to(jnp.sum(mask, axis=0)[:, None], (G, D))

        @pl.when(i == _NB - 1)
        def _():
            o_ref[...] = sums[...] / jnp.maximum(cnts[...], 1.0)

    return pl.pallas_call(
        body,
        grid=(_NB,),
        in_specs=[pl.BlockSpec((_RB, D), lambda i: (i, 0)),
                  pl.BlockSpec((1, 1, _RB), lambda i: (i, 0, 0))],
        out_specs=pl.BlockSpec((G, D), lambda i: (0, 0)),
        out_shape=jax.ShapeDtypeStruct((G, D), _f32),
        scratch_shapes=[pltpu.VMEM((G, D), _f32), pltpu.VMEM((G, D), _f32)],
    )(h, batch3d)


def kernel(x, hyperedge_index, batch, W1, b1, W2, b2):
    node_idx = hyperedge_index[0].astype(jnp.int32)
    edge_idx = hyperedge_index[1].astype(jnp.int32)
    batch3d = batch.astype(jnp.int32).reshape(_NB, 1, _RB)

    n_pad = NI_PAD - NI
    src_pad = jnp.arange(n_pad, dtype=jnp.int32) % N
    dst_pad = PAD_ROW + (jnp.arange(n_pad, dtype=jnp.int32) % (NP - N))
    node_src = jnp.concatenate([node_idx, src_pad])
    node_dst = jnp.concatenate([node_idx, dst_pad])
    edge_src = jnp.concatenate([edge_idx, src_pad])
    edge_dst = jnp.concatenate([edge_idx, dst_pad])

    zeros_nd = jnp.zeros((NP, DH), _f32)
    zeros_cw = jnp.zeros((NP, CW), _f32)

    xwL, xwR = _tc_matmul(x, W1)
    outP, cntD = _layer(xwL, xwR, node_src, edge_dst, edge_src, node_dst,
                        zeros_nd, zeros_cw)
    h = _tc_combine(outP, cntD, b1)

    xwL, xwR = _tc_matmul(h, W2)
    outP, cntD = _layer(xwL, xwR, node_src, edge_dst, edge_src, node_dst,
                        zeros_nd, zeros_cw)
    h = _tc_combine(outP, cntD, b2)

    return _tc_pool(h, batch3d)


# speedup vs baseline: 1.4258x; 1.3860x over previous
import jax
import jax.numpy as jnp
from jax import lax
from jax.experimental import pallas as pl
from jax.experimental.pallas import tpu as pltpu
from jax.experimental.pallas import tpu_sc as plsc

N = 10000
NI = 320000
D = 128
G = 64
L = 16

NC = 2
NS = 16
NW = NC * NS
CHUNK = 128
N_CHUNKS = 80
PER_TILE = CHUNK * N_CHUNKS
NI_PAD = PER_TILE * NW
NP = 10240
PAD_ROW = N
ROWS_PER_TILE = NP // NS

_MESH = plsc.VectorSubcoreMesh(core_axis_name="c", subcore_axis_name="s")

_f32 = jnp.float32


def _phase_body(table, src, dst, zeros_nd, zeros_np,
                out, cnt_out,
                sv0, sv1, sv2, sv3, dv0, dv1, dv2, dv3, rv0, rv1, hist,
                acc_sh,
                si0, si1, si2, si3, sg0, sg1, ss0, ss1):
    src_v = (sv0, sv1, sv2, sv3)
    dst_v = (dv0, dv1, dv2, dv3)
    rows_v = (rv0, rv1)
    sem_i = (si0, si1, si2, si3)
    sem_g = (sg0, sg1)
    sem_s = (ss0, ss1)

    cid = lax.axis_index("c")
    sid = lax.axis_index("s")
    wid = cid * NS + sid

    @pl.when(sid == 0)
    def _():
        pltpu.sync_copy(zeros_nd, acc_sh)

    pltpu.sync_copy(zeros_np, hist)
    plsc.subcore_barrier()

    ones16 = jnp.ones((L,), _f32)

    def issue_idx(j, c):
        base = wid * PER_TILE + c * CHUNK
        pltpu.async_copy(src.at[pl.ds(base, CHUNK)], src_v[j], sem_i[j])
        pltpu.async_copy(dst.at[pl.ds(base, CHUNK)], dst_v[j], sem_i[j])

    def wait_idx(j):
        pltpu.make_async_copy(src.at[pl.ds(0, CHUNK)], src_v[j], sem_i[j]).wait()
        pltpu.make_async_copy(dst.at[pl.ds(0, CHUNK)], dst_v[j], sem_i[j]).wait()

    def issue_scatter(j, b):
        pltpu.async_copy(rows_v[b], acc_sh.at[dst_v[j]], sem_s[b], add=True)

    def wait_scatter(j, b):
        pltpu.make_async_copy(rows_v[b], acc_sh.at[dst_v[j]], sem_s[b]).wait()

    def wait_gather(j, b):
        pltpu.make_async_copy(table.at[src_v[j]], rows_v[b], sem_g[b]).wait()

    def hist_update(j):
        for k in range(CHUNK // L):
            idx16 = dst_v[j][pl.ds(k * L, L)]
            plsc.addupdate_scatter(hist, [idx16], ones16)

    issue_idx(0, 0)
    issue_idx(1, 1)

    def body(s, carry):
        for j in range(4):
            c = 4 * s + j
            b = j % 2
            wait_idx(j)

            @pl.when(c >= 2)
            def _(j=j, b=b):
                wait_scatter((j + 2) % 4, b)

            @pl.when(c + 2 < N_CHUNKS)
            def _(j=j, c=c):
                issue_idx((j + 2) % 4, c + 2)

            pltpu.async_copy(table.at[src_v[j]], rows_v[b], sem_g[b])
            hist_update(j)

            @pl.when(c >= 1)
            def _(j=j, b=b):
                wait_gather((j + 3) % 4, 1 - b)
                issue_scatter((j + 3) % 4, 1 - b)

        return carry

    lax.fori_loop(0, N_CHUNKS // 4, body, 0)

    j_last = (N_CHUNKS - 1) % 4
    b_last = (N_CHUNKS - 1) % 2
    wait_gather(j_last, b_last)
    issue_scatter(j_last, b_last)
    wait_scatter((N_CHUNKS - 2) % 4, (N_CHUNKS - 2) % 2)
    wait_scatter(j_last, b_last)

    plsc.subcore_barrier()

    r0 = sid * ROWS_PER_TILE

    def wb(k, carry):
        pltpu.sync_copy(acc_sh.at[pl.ds(r0 + k * CHUNK, CHUNK)], rows_v[0])
        pltpu.sync_copy(rows_v[0],
                        out.at[pl.ds(cid * NP + r0 + k * CHUNK, CHUNK)])
        return carry

    lax.fori_loop(0, ROWS_PER_TILE // CHUNK, wb, 0)
    pltpu.sync_copy(hist, cnt_out.at[wid])


_phase = pl.kernel(
    _phase_body,
    out_type=(
        jax.ShapeDtypeStruct((NC * NP, D), _f32),
        jax.ShapeDtypeStruct((NW, NP), _f32),
    ),
    mesh=_MESH,
    scratch_types=(
        [pltpu.VMEM((CHUNK,), jnp.int32)] * 8
        + [pltpu.VMEM((CHUNK, D), _f32)] * 2
        + [pltpu.VMEM((NP,), _f32)]
        + [pltpu.VMEM_SHARED((NP, D), _f32)]
        + [pltpu.SemaphoreType.DMA] * 8
    ),
    compiler_params=pltpu.CompilerParams(use_tc_tiling_on_sc=False,
                                         needs_layout_passes=False),
)


_RB = 1000
_NB = N // _RB


def _tc_matmul(x, W):
    def body(x_ref, w_ref, o_ref):
        o_ref[...] = jnp.dot(x_ref[...], w_ref[...],
                             preferred_element_type=_f32)

    return pl.pallas_call(
        body,
        grid=(_NB,),
        in_specs=[pl.BlockSpec((_RB, D), lambda i: (i, 0)),
                  pl.BlockSpec((D, D), lambda i: (0, 0))],
        out_specs=pl.BlockSpec((_RB, D), lambda i: (i, 0)),
        out_shape=jax.ShapeDtypeStruct((N, D), _f32),
    )(x, W)


def _combine_block(p_ref, c_ref):
    s = p_ref[0] + p_ref[1]
    cnt = jnp.sum(c_ref[:, 0, 0, :], axis=0)[:, None]
    inv = jnp.where(cnt > 0.0, 1.0 / cnt, 0.0)
    return s * inv


def _tc_combine(partials, cnts):
    p3 = partials.reshape(NC, NP, D)
    cnts = cnts[:, :N].reshape(NW, _NB, 1, _RB)

    def body(p_ref, c_ref, o_ref):
        o_ref[...] = _combine_block(p_ref, c_ref)

    return pl.pallas_call(
        body,
        grid=(_NB,),
        in_specs=[pl.BlockSpec((NC, _RB, D), lambda i: (0, i, 0)),
                  pl.BlockSpec((NW, 1, 1, _RB), lambda i: (0, i, 0, 0))],
        out_specs=pl.BlockSpec((_RB, D), lambda i: (i, 0)),
        out_shape=jax.ShapeDtypeStruct((N, D), _f32),
    )(p3, cnts)


def _tc_combine_relu_mm(partials, cnts, bias, W):
    p3 = partials.reshape(NC, NP, D)
    cnts = cnts[:, :N].reshape(NW, _NB, 1, _RB)

    def body(p_ref, c_ref, b_ref, w_ref, o_ref):
        h = jnp.maximum(
            _combine_block(p_ref, c_ref) + b_ref[...], 0.0)
        o_ref[...] = jnp.dot(h, w_ref[...], preferred_element_type=_f32)

    return pl.pallas_call(
        body,
        grid=(_NB,),
        in_specs=[pl.BlockSpec((NC, _RB, D), lambda i: (0, i, 0)),
                  pl.BlockSpec((NW, 1, 1, _RB), lambda i: (0, i, 0, 0)),
                  pl.BlockSpec((1, D), lambda i: (0, 0)),
                  pl.BlockSpec((D, D), lambda i: (0, 0))],
        out_specs=pl.BlockSpec((_RB, D), lambda i: (i, 0)),
        out_shape=jax.ShapeDtypeStruct((N, D), _f32),
    )(p3, cnts, bias.reshape(1, D), W)


def _tc_combine_relu_pool(partials, cnts, bias, batch3d):
    p3 = partials.reshape(NC, NP, D)
    cnts = cnts[:, :N].reshape(NW, _NB, 1, _RB)

    def body(p_ref, c_ref, b_ref, bt_ref, o_ref, sums, pcnts):
        i = pl.program_id(0)

        @pl.when(i == 0)
        def _():
            sums[...] = jnp.zeros_like(sums)
            pcnts[...] = jnp.zeros_like(pcnts)

        h = jnp.maximum(_combine_block(p_ref, c_ref) + b_ref[...], 0.0)
        b = bt_ref[0, 0, :]
        mask = (b[:, None] == lax.broadcasted_iota(jnp.int32, (_RB, G), 1)
                ).astype(_f32)
        sums[...] += lax.dot_general(mask, h, (((0,), (0,)), ((), ())),
                                     preferred_element_type=_f32)
        pcnts[...] += jnp.broadcast_to(jnp.sum(mask, axis=0)[:, None], (G, D))

        @pl.when(i == _NB - 1)
        def _():
            o_ref[...] = sums[...] / jnp.maximum(pcnts[...], 1.0)

    return pl.pallas_call(
        body,
        grid=(_NB,),
        in_specs=[pl.BlockSpec((NC, _RB, D), lambda i: (0, i, 0)),
                  pl.BlockSpec((NW, 1, 1, _RB), lambda i: (0, i, 0, 0)),
                  pl.BlockSpec((1, D), lambda i: (0, 0)),
                  pl.BlockSpec((1, 1, _RB), lambda i: (i, 0, 0))],
        out_specs=pl.BlockSpec((G, D), lambda i: (0, 0)),
        out_shape=jax.ShapeDtypeStruct((G, D), _f32),
        scratch_shapes=[pltpu.VMEM((G, D), _f32), pltpu.VMEM((G, D), _f32)],
    )(p3, cnts, bias.reshape(1, D), batch3d)


def kernel(x, hyperedge_index, batch, W1, b1, W2, b2):
    node_idx = hyperedge_index[0].astype(jnp.int32)
    edge_idx = hyperedge_index[1].astype(jnp.int32)
    batch3d = batch.astype(jnp.int32).reshape(_NB, 1, _RB)

    n_pad = NI_PAD - NI
    src_pad = jnp.arange(n_pad, dtype=jnp.int32) % N
    dst_pad = PAD_ROW + (jnp.arange(n_pad, dtype=jnp.int32) % (NP - N))
    node_src = jnp.concatenate([node_idx, src_pad])
    node_dst = jnp.concatenate([node_idx, dst_pad])
    edge_src = jnp.concatenate([edge_idx, src_pad])
    edge_dst = jnp.concatenate([edge_idx, dst_pad])

    zeros_nd = jnp.zeros((NP, D), _f32)
    zeros_np = jnp.zeros((NP,), _f32)

    xw = _tc_matmul(x, W1)
    heP, cntB = _phase(xw, node_src, edge_dst, zeros_nd, zeros_np)
    he = _tc_combine(heP, cntB)
    outP, cntD = _phase(he, edge_src, node_dst, zeros_nd, zeros_np)
    xw = _tc_combine_relu_mm(outP, cntD, b1, W2)

    heP, cntB = _phase(xw, node_src, edge_dst, zeros_nd, zeros_np)
    he = _tc_combine(heP, cntB)
    outP, cntD = _phase(he, edge_src, node_dst, zeros_nd, zeros_np)
    return _tc_combine_relu_pool(outP, cntD, b2, batch3d)


# speedup vs baseline: 1.4488x; 1.0161x over previous
import jax
import jax.numpy as jnp
from jax import lax
from jax.experimental import pallas as pl
from jax.experimental.pallas import tpu as pltpu
from jax.experimental.pallas import tpu_sc as plsc

N = 10000
NI = 320000
D = 128
G = 64
L = 16

NC = 2
NS = 16
NW = NC * NS
CHUNK = 128
N_CHUNKS = 80
PER_TILE = CHUNK * N_CHUNKS
NI_PAD = PER_TILE * NW
NP = 10240
PAD_ROW = N
ROWS_PER_TILE = NP // NS

_MESH = plsc.VectorSubcoreMesh(core_axis_name="c", subcore_axis_name="s")

_f32 = jnp.float32


def _phase_body(table, src, dst, zeros_nd, zeros_np,
                out, cnt_out,
                sv0, sv1, sv2, sv3, dv0, dv1, dv2, dv3, rv0, rv1, hist,
                acc_sh,
                si0, si1, si2, si3, sg0, sg1, ss0, ss1):
    src_v = (sv0, sv1, sv2, sv3)
    dst_v = (dv0, dv1, dv2, dv3)
    rows_v = (rv0, rv1)
    sem_i = (si0, si1, si2, si3)
    sem_g = (sg0, sg1)
    sem_s = (ss0, ss1)

    cid = lax.axis_index("c")
    sid = lax.axis_index("s")
    wid = cid * NS + sid

    r0 = sid * ROWS_PER_TILE
    pltpu.sync_copy(zeros_nd.at[pl.ds(r0, ROWS_PER_TILE)],
                    acc_sh.at[pl.ds(r0, ROWS_PER_TILE)])
    pltpu.sync_copy(zeros_np, hist)
    plsc.subcore_barrier()

    ones16 = jnp.ones((L,), _f32)

    def issue_idx(j, c):
        base = wid * PER_TILE + c * CHUNK
        pltpu.async_copy(src.at[pl.ds(base, CHUNK)], src_v[j], sem_i[j])
        pltpu.async_copy(dst.at[pl.ds(base, CHUNK)], dst_v[j], sem_i[j])

    def wait_idx(j):
        pltpu.make_async_copy(src.at[pl.ds(0, CHUNK)], src_v[j], sem_i[j]).wait()
        pltpu.make_async_copy(dst.at[pl.ds(0, CHUNK)], dst_v[j], sem_i[j]).wait()

    def issue_scatter(j, b):
        pltpu.async_copy(rows_v[b], acc_sh.at[dst_v[j]], sem_s[b], add=True)

    def wait_scatter(j, b):
        pltpu.make_async_copy(rows_v[b], acc_sh.at[dst_v[j]], sem_s[b]).wait()

    def wait_gather(j, b):
        pltpu.make_async_copy(table.at[src_v[j]], rows_v[b], sem_g[b]).wait()

    def hist_update(j):
        for k in range(CHUNK // L):
            idx16 = dst_v[j][pl.ds(k * L, L)]
            plsc.addupdate_scatter(hist, [idx16], ones16)

    issue_idx(0, 0)
    issue_idx(1, 1)

    def body(s, carry):
        for j in range(4):
            c = 4 * s + j
            b = j % 2
            wait_idx(j)

            @pl.when(c >= 2)
            def _(j=j, b=b):
                wait_scatter((j + 2) % 4, b)

            @pl.when(c + 2 < N_CHUNKS)
            def _(j=j, c=c):
                issue_idx((j + 2) % 4, c + 2)

            pltpu.async_copy(table.at[src_v[j]], rows_v[b], sem_g[b])
            hist_update(j)

            @pl.when(c >= 1)
            def _(j=j, b=b):
                wait_gather((j + 3) % 4, 1 - b)
                issue_scatter((j + 3) % 4, 1 - b)

        return carry

    lax.fori_loop(0, N_CHUNKS // 4, body, 0)

    j_last = (N_CHUNKS - 1) % 4
    b_last = (N_CHUNKS - 1) % 2
    wait_gather(j_last, b_last)
    issue_scatter(j_last, b_last)
    wait_scatter((N_CHUNKS - 2) % 4, (N_CHUNKS - 2) % 2)
    wait_scatter(j_last, b_last)

    plsc.subcore_barrier()

    hist_cp = pltpu.async_copy(hist, cnt_out.at[wid], sem_s[0])
    descs = {}
    for k in range(ROWS_PER_TILE // CHUNK):
        b = k % 2
        if k >= 2:
            descs[k - 2].wait()
        pltpu.sync_copy(acc_sh.at[pl.ds(r0 + k * CHUNK, CHUNK)], rows_v[b])
        descs[k] = pltpu.async_copy(
            rows_v[b], out.at[pl.ds(cid * NP + r0 + k * CHUNK, CHUNK)],
            sem_g[b])
    descs[ROWS_PER_TILE // CHUNK - 2].wait()
    descs[ROWS_PER_TILE // CHUNK - 1].wait()
    hist_cp.wait()


_phase = pl.kernel(
    _phase_body,
    out_type=(
        jax.ShapeDtypeStruct((NC * NP, D), _f32),
        jax.ShapeDtypeStruct((NW, NP), _f32),
    ),
    mesh=_MESH,
    scratch_types=(
        [pltpu.VMEM((CHUNK,), jnp.int32)] * 8
        + [pltpu.VMEM((CHUNK, D), _f32)] * 2
        + [pltpu.VMEM((NP,), _f32)]
        + [pltpu.VMEM_SHARED((NP, D), _f32)]
        + [pltpu.SemaphoreType.DMA] * 8
    ),
    compiler_params=pltpu.CompilerParams(use_tc_tiling_on_sc=False,
                                         needs_layout_passes=False),
)


_RB = 1000
_NB = N // _RB


def _tc_matmul(x, W):
    def body(x_ref, w_ref, o_ref):
        o_ref[...] = jnp.dot(x_ref[...], w_ref[...],
                             preferred_element_type=_f32)

    return pl.pallas_call(
        body,
        grid=(_NB,),
        in_specs=[pl.BlockSpec((_RB, D), lambda i: (i, 0)),
                  pl.BlockSpec((D, D), lambda i: (0, 0))],
        out_specs=pl.BlockSpec((_RB, D), lambda i: (i, 0)),
        out_shape=jax.ShapeDtypeStruct((N, D), _f32),
    )(x, W)


def _combine_block(p_ref, c_ref):
    s = p_ref[0] + p_ref[1]
    cnt = jnp.sum(c_ref[:, 0, 0, :], axis=0)[:, None]
    inv = jnp.where(cnt > 0.0, 1.0 / cnt, 0.0)
    return s * inv


def _tc_combine(partials, cnts):
    p3 = partials.reshape(NC, NP, D)
    cnts = cnts[:, :N].reshape(NW, _NB, 1, _RB)

    def body(p_ref, c_ref, o_ref):
        o_ref[...] = _combine_block(p_ref, c_ref)

    return pl.pallas_call(
        body,
        grid=(_NB,),
        in_specs=[pl.BlockSpec((NC, _RB, D), lambda i: (0, i, 0)),
                  pl.BlockSpec((NW, 1, 1, _RB), lambda i: (0, i, 0, 0))],
        out_specs=pl.BlockSpec((_RB, D), lambda i: (i, 0)),
        out_shape=jax.ShapeDtypeStruct((N, D), _f32),
    )(p3, cnts)


def _tc_combine_relu_mm(partials, cnts, bias, W):
    p3 = partials.reshape(NC, NP, D)
    cnts = cnts[:, :N].reshape(NW, _NB, 1, _RB)

    def body(p_ref, c_ref, b_ref, w_ref, o_ref):
        h = jnp.maximum(
            _combine_block(p_ref, c_ref) + b_ref[...], 0.0)
        o_ref[...] = jnp.dot(h, w_ref[...], preferred_element_type=_f32)

    return pl.pallas_call(
        body,
        grid=(_NB,),
        in_specs=[pl.BlockSpec((NC, _RB, D), lambda i: (0, i, 0)),
                  pl.BlockSpec((NW, 1, 1, _RB), lambda i: (0, i, 0, 0)),
                  pl.BlockSpec((1, D), lambda i: (0, 0)),
                  pl.BlockSpec((D, D), lambda i: (0, 0))],
        out_specs=pl.BlockSpec((_RB, D), lambda i: (i, 0)),
        out_shape=jax.ShapeDtypeStruct((N, D), _f32),
    )(p3, cnts, bias.reshape(1, D), W)


def _tc_combine_relu_pool(partials, cnts, bias, batch3d):
    p3 = partials.reshape(NC, NP, D)
    cnts = cnts[:, :N].reshape(NW, _NB, 1, _RB)

    def body(p_ref, c_ref, b_ref, bt_ref, o_ref, sums, pcnts):
        i = pl.program_id(0)

        @pl.when(i == 0)
        def _():
            sums[...] = jnp.zeros_like(sums)
            pcnts[...] = jnp.zeros_like(pcnts)

        h = jnp.maximum(_combine_block(p_ref, c_ref) + b_ref[...], 0.0)
        b = bt_ref[0, 0, :]
        mask = (b[:, None] == lax.broadcasted_iota(jnp.int32, (_RB, G), 1)
                ).astype(_f32)
        sums[...] += lax.dot_general(mask, h, (((0,), (0,)), ((), ())),
                                     preferred_element_type=_f32)
        pcnts[...] += jnp.broadcast_to(jnp.sum(mask, axis=0)[:, None], (G, D))

        @pl.when(i == _NB - 1)
        def _():
            o_ref[...] = sums[...] / jnp.maximum(pcnts[...], 1.0)

    return pl.pallas_call(
        body,
        grid=(_NB,),
        in_specs=[pl.BlockSpec((NC, _RB, D), lambda i: (0, i, 0)),
                  pl.BlockSpec((NW, 1, 1, _RB), lambda i: (0, i, 0, 0)),
                  pl.BlockSpec((1, D), lambda i: (0, 0)),
                  pl.BlockSpec((1, 1, _RB), lambda i: (i, 0, 0))],
        out_specs=pl.BlockSpec((G, D), lambda i: (0, 0)),
        out_shape=jax.ShapeDtypeStruct((G, D), _f32),
        scratch_shapes=[pltpu.VMEM((G, D), _f32), pltpu.VMEM((G, D), _f32)],
    )(p3, cnts, bias.reshape(1, D), batch3d)


def kernel(x, hyperedge_index, batch, W1, b1, W2, b2):
    node_idx = hyperedge_index[0].astype(jnp.int32)
    edge_idx = hyperedge_index[1].astype(jnp.int32)
    batch3d = batch.astype(jnp.int32).reshape(_NB, 1, _RB)

    n_pad = NI_PAD - NI
    src_pad = jnp.arange(n_pad, dtype=jnp.int32) % N
    dst_pad = PAD_ROW + (jnp.arange(n_pad, dtype=jnp.int32) % (NP - N))
    node_src = jnp.concatenate([node_idx, src_pad])
    node_dst = jnp.concatenate([node_idx, dst_pad])
    edge_src = jnp.concatenate([edge_idx, src_pad])
    edge_dst = jnp.concatenate([edge_idx, dst_pad])

    zeros_nd = jnp.zeros((NP, D), _f32)
    zeros_np = jnp.zeros((NP,), _f32)

    xw = _tc_matmul(x, W1)
    heP, cntB = _phase(xw, node_src, edge_dst, zeros_nd, zeros_np)
    he = _tc_combine(heP, cntB)
    outP, cntD = _phase(he, edge_src, node_dst, zeros_nd, zeros_np)
    xw = _tc_combine_relu_mm(outP, cntD, b1, W2)

    heP, cntB = _phase(xw, node_src, edge_dst, zeros_nd, zeros_np)
    he = _tc_combine(heP, cntB)
    outP, cntD = _phase(he, edge_src, node_dst, zeros_nd, zeros_np)
    return _tc_combine_relu_pool(outP, cntD, b2, batch3d)
